# baseline (speedup 1.0000x reference)
import functools

import jax
import jax.numpy as jnp
from jax import lax
from jax.experimental import pallas as pl
from jax.experimental.pallas import tpu as pltpu
from jax.experimental.pallas import tpu_sc as plsc

N = 50000
NP = 50176
E = 800000
EP = 819200
H = 64
HH = 32
G = 512
GP = 528
RPT = NP // 16
EPT = EP // 16
C = 512
NCH = EPT // C
RB = 1024
NBLK = NP // RB


def _flat_spec():
    return pl.BlockSpec((8, 128), lambda i: (i, 0))


def _attention_rows(h, av, ad):
    asr = jnp.sum(h * av, axis=-1)
    ads = jnp.sum(h * ad, axis=-1)
    t = asr + ads
    asel = jnp.where(t >= 0, t, 0.2 * t)
    return asr, ads, asel


def _dense1_body(x_ref, w_ref, av_ref, ad_ref,
                 hA_ref, hB_ref, as_ref, ad2_ref, asel_ref):
    h = jnp.dot(x_ref[...], w_ref[...], preferred_element_type=jnp.float32)
    hA_ref[...] = h[:, :HH]
    hB_ref[...] = h[:, HH:]
    asr, ads, asel = _attention_rows(h, av_ref[...], ad_ref[...])
    as_ref[...] = asr.reshape(8, 128)
    ad2_ref[...] = ads.reshape(8, 128)
    asel_ref[...] = asel.reshape(8, 128)


def _combine_body(relu, accA_ref, accB_ref, den0_ref, den1_ref,
                  hA_ref, hB_ref, b_ref,
                  w_ref, av_ref, ad_ref,
                  hA2_ref, hB2_ref, as_ref, ad2_ref, asel_ref):
    r = 1.0 / (den0_ref[...] + den1_ref[...] + 1.0)
    xin = jnp.concatenate(
        [accA_ref[...] + hA_ref[...], accB_ref[...] + hB_ref[...]], axis=1)
    xin = xin * r + b_ref[...]
    if relu:
        xin = jnp.maximum(xin, 0.0)
    h = jnp.dot(xin, w_ref[...], preferred_element_type=jnp.float32)
    hA2_ref[...] = h[:, :HH]
    hB2_ref[...] = h[:, HH:]
    asr, ads, asel = _attention_rows(h, av_ref[...], ad_ref[...])
    as_ref[...] = asr.reshape(8, 128)
    ad2_ref[...] = ads.reshape(8, 128)
    asel_ref[...] = asel.reshape(8, 128)


def _final_combine_body(accA_ref, accB_ref, den0_ref, den1_ref,
                        hA_ref, hB_ref, b_ref, x3_ref):
    r = 1.0 / (den0_ref[...] + den1_ref[...] + 1.0)
    xin = jnp.concatenate(
        [accA_ref[...] + hA_ref[...], accB_ref[...] + hB_ref[...]], axis=1)
    x3_ref[...] = xin * r + b_ref[...]


def _head_body(sums_ref, cnt_ref, wl_ref, bl_ref, out_ref):
    s = sums_ref[0] + sums_ref[1]
    c = cnt_ref[0] + cnt_ref[1]
    pooled = s[:G] / jnp.maximum(c[:G], 1.0)[:, None]
    out_ref[...] = (
        jnp.dot(pooled, wl_ref[...], preferred_element_type=jnp.float32)
        + bl_ref[...])


_half_spec = pl.BlockSpec((RB, HH), lambda i: (i, 0))
_full_row_spec = pl.BlockSpec((RB, H), lambda i: (i, 0))
_vec64_spec = pl.BlockSpec((1, H), lambda i: (0, 0))
_den_spec = pl.BlockSpec((RB, 1), lambda i: (i, 0))

_f32 = jnp.float32


def _shape_f(shape):
    return jax.ShapeDtypeStruct(shape, _f32)


_dense1 = pl.pallas_call(
    _dense1_body,
    grid=(NBLK,),
    in_specs=[
        pl.BlockSpec((RB, 24), lambda i: (i, 0)),
        pl.BlockSpec((24, H), lambda i: (0, 0)),
        _vec64_spec, _vec64_spec,
    ],
    out_specs=[_half_spec, _half_spec, _flat_spec(), _flat_spec(), _flat_spec()],
    out_shape=[_shape_f((NP, HH)), _shape_f((NP, HH)),
               _shape_f((392, 128)), _shape_f((392, 128)), _shape_f((392, 128))],
)


def _make_combine(relu):
    return pl.pallas_call(
        functools.partial(_combine_body, relu),
        grid=(NBLK,),
        in_specs=[
            _half_spec, _half_spec, _den_spec, _den_spec,
            _half_spec, _half_spec, _vec64_spec,
            pl.BlockSpec((H, H), lambda i: (0, 0)),
            _vec64_spec, _vec64_spec,
        ],
        out_specs=[_half_spec, _half_spec, _flat_spec(), _flat_spec(),
                   _flat_spec()],
        out_shape=[_shape_f((NP, HH)), _shape_f((NP, HH)),
                   _shape_f((392, 128)), _shape_f((392, 128)),
                   _shape_f((392, 128))],
    )


_combine_relu = _make_combine(True)

_final_combine = pl.pallas_call(
    _final_combine_body,
    grid=(NBLK,),
    in_specs=[
        _half_spec, _half_spec, _den_spec, _den_spec,
        _half_spec, _half_spec, _vec64_spec,
    ],
    out_specs=_full_row_spec,
    out_shape=_shape_f((NP, H)),
)

_head = pl.pallas_call(
    _head_body,
    grid=(1,),
    in_specs=[
        pl.BlockSpec((2, GP, H), lambda i: (0, 0, 0)),
        pl.BlockSpec((2, GP), lambda i: (0, 0)),
        pl.BlockSpec((H, 2), lambda i: (0, 0)),
        pl.BlockSpec((1, 2), lambda i: (0, 0)),
    ],
    out_specs=pl.BlockSpec((G, 2), lambda i: (0, 0)),
    out_shape=_shape_f((G, 2)),
)


_i32 = jnp.int32
_lane16 = jnp.arange(16, dtype=jnp.int32)


def _edge_body(src1, dst2, asrc, adst, aself, hA, hB,
               accA_o, accB_o, den_o,
               src_a, src_b, d2_a, d2_b, av, bv, cv, s1, h_v, zrows, zflat,
               acc_sp, den_sp, sem_ld, sem_g, sem_h, sem_sc):
    cid = lax.axis_index("c")
    wid = lax.axis_index("s")

    @plsc.parallel_loop(0, 64)
    def _(i):
        z = jnp.zeros((16,), _f32)
        zrows[i, pl.ds(0, 16)] = z
        zrows[i, pl.ds(16, 16)] = z

    @plsc.parallel_loop(0, RPT // 16)
    def _(i):
        zflat[pl.ds(i * 16, 16)] = jnp.zeros((16,), _f32)

    row0 = wid * RPT
    for k in range(RPT // 64):
        pltpu.sync_copy(zrows, acc_sp.at[pl.ds(row0 + k * 64, 64)])
    pltpu.sync_copy(zflat, den_sp.at[pl.ds(row0, RPT)])
    plsc.subcore_barrier()

    def jbase(j):
        return pl.multiple_of(wid * EPT + j * C, C)

    def rbase(j):
        return wid * (EPT // 128) + j * (C // 128)

    def drain_scatters(dp, d2_c):
        for jj in range(C // 128):
            pltpu.make_async_copy(h_v.at[pl.ds(jj * 128, 128)],
                                  acc_sp.at[d2_c.at[jj]], sem_sc).wait()

        @pl.when(cid == dp)
        def _():
            for jj in range(C // 128):
                pltpu.make_async_copy(s1.at[pl.ds(jj * 128, 128)],
                                      den_sp.at[d2_c.at[jj]], sem_sc).wait()

    def phase(j, src_c, d2_c, src_n, d2_n, first):
        pltpu.make_async_copy(src1.at[pl.ds(jbase(j), C)], src_c,
                              sem_ld).wait()
        pltpu.make_async_copy(dst2.at[pl.ds(rbase(j), C // 128)], d2_c,
                              sem_ld).wait()
        gw = [pltpu.async_copy(asrc.at[src_c], av, sem_g)]
        for jj in range(C // 128):
            idx = d2_c.at[jj]
            gw.append(pltpu.async_copy(adst.at[idx], bv.at[jj], sem_g))
            gw.append(pltpu.async_copy(aself.at[idx], cv.at[jj], sem_g))

        if first:
            @pl.when(j > 0)
            def _():
                drain_scatters(1, d2_c)
        else:
            drain_scatters(0, d2_c)

        @pl.when(cid == 0)
        def _():
            pltpu.async_copy(hA.at[src_c], h_v, sem_h)

        @pl.when(cid == 1)
        def _():
            pltpu.async_copy(hB.at[src_c], h_v, sem_h)

        @pl.when(j + 1 < NCH)
        def _():
            pltpu.async_copy(src1.at[pl.ds(jbase(j + 1), C)], src_n, sem_ld)
            pltpu.async_copy(dst2.at[pl.ds(rbase(j + 1), C // 128)], d2_n,
                             sem_ld)

        for wcp in gw:
            wcp.wait()

        @plsc.parallel_loop(0, C // 16)
        def _(g):
            a16 = av[pl.ds(g * 16, 16)]
            b16 = bv[g // 8, pl.ds((g % 8) * 16, 16)]
            c16 = cv[g // 8, pl.ds((g % 8) * 16, 16)]
            al = a16 + b16
            al = jnp.where(al >= 0, al, 0.2 * al)
            s1[pl.ds(g * 16, 16)] = jnp.exp(jnp.minimum(al - c16, 60.0))

        pltpu.make_async_copy(hA.at[src_c], h_v, sem_h).wait()

        @plsc.parallel_loop(0, C // 16, unroll=2)
        def _(g):
            sval = s1[pl.ds(g * 16, 16)]
            for k in range(16):
                e = g * 16 + k
                sv = sval[k]
                h_v[e, pl.ds(0, 16)] = h_v[e, pl.ds(0, 16)] * sv
                h_v[e, pl.ds(16, 16)] = h_v[e, pl.ds(16, 16)] * sv

        for jj in range(C // 128):
            pltpu.async_copy(h_v.at[pl.ds(jj * 128, 128)],
                             acc_sp.at[d2_c.at[jj]], sem_sc, add=True)

        @pl.when(cid == (0 if first else 1))
        def _():
            for jj in range(C // 128):
                pltpu.async_copy(s1.at[pl.ds(jj * 128, 128)],
                                 den_sp.at[d2_c.at[jj]], sem_sc, add=True)

    pltpu.async_copy(src1.at[pl.ds(jbase(0), C)], src_a, sem_ld)
    pltpu.async_copy(dst2.at[pl.ds(rbase(0), C // 128)], d2_a, sem_ld)

    def pair(t, _):
        phase(2 * t, src_a, d2_a, src_b, d2_b, True)
        phase(2 * t + 1, src_b, d2_b, src_a, d2_a, False)
        return ()

    lax.fori_loop(0, NCH // 2, pair, ())
    drain_scatters(1, d2_a)
    plsc.subcore_barrier()

    rows = pl.ds(row0, RPT)

    @pl.when(cid == 0)
    def _():
        pltpu.sync_copy(acc_sp.at[rows], accA_o.at[rows])

    @pl.when(cid == 1)
    def _():
        pltpu.sync_copy(acc_sp.at[rows], accB_o.at[rows])

    pltpu.sync_copy(den_sp.at[rows], den_o.at[cid, rows])


@functools.cache
def _edge_kernel():
  return pl.kernel(
    _edge_body,
    out_type=[_shape_f((NP, HH)), _shape_f((NP, HH)), _shape_f((2, NP))],
    mesh=plsc.VectorSubcoreMesh(core_axis_name="c", subcore_axis_name="s"),
    compiler_params=pltpu.CompilerParams(use_tc_tiling_on_sc=False),
    scratch_types=[
        pltpu.VMEM((C,), _i32),
        pltpu.VMEM((C,), _i32),
        pltpu.VMEM((C // 128, 128), _i32),
        pltpu.VMEM((C // 128, 128), _i32),
        pltpu.VMEM((C,), _f32),
        pltpu.VMEM((C // 128, 128), _f32),
        pltpu.VMEM((C // 128, 128), _f32),
        pltpu.VMEM((C,), _f32),
        pltpu.VMEM((C, HH), _f32),
        pltpu.VMEM((64, HH), _f32),
        pltpu.VMEM((RPT,), _f32),
        pltpu.VMEM_SHARED((NP, HH), _f32),
        pltpu.VMEM_SHARED((NP,), _f32),
        pltpu.SemaphoreType.DMA,
        pltpu.SemaphoreType.DMA,
        pltpu.SemaphoreType.DMA,
        pltpu.SemaphoreType.DMA,
    ],
  )


PRT = NP // 32
PCH = 224
PSC = 32


def _pool_body(x3, b2, sums_o, cnt_o,
               bidx_v, xbuf, ones_v, zrow, zc, sums_sp, cnt_sp, sem):
    cid = lax.axis_index("c")
    sid = lax.axis_index("s")
    wid = cid * 16 + sid

    @plsc.parallel_loop(0, 2)
    def _(i):
        ones_v[pl.ds(i * 16, 16)] = jnp.full((16,), 1.0, _f32)

    @pl.when(sid == 0)
    def _():
        @plsc.parallel_loop(0, 33 * 4)
        def _(i):
            zrow[i // 4, pl.ds((i % 4) * 16, 16)] = jnp.zeros((16,), _f32)

        @plsc.parallel_loop(0, GP // 16)
        def _(i):
            zc[pl.ds(i * 16, 16)] = jnp.zeros((16,), _f32)

        for k in range(GP // 33):
            pltpu.sync_copy(zrow, sums_sp.at[pl.ds(k * 33, 33)])
        pltpu.sync_copy(zc, cnt_sp)

    plsc.subcore_barrier()

    pltpu.sync_copy(b2.at[pl.ds(wid * (PRT // PSC), PRT // PSC)], bidx_v)

    def piter(k, _):
        base = pl.multiple_of(wid * PRT + k * PCH, PCH)
        pltpu.sync_copy(x3.at[pl.ds(base, PCH)], xbuf)
        for m in range(PCH // PSC):
            idx = bidx_v.at[k * (PCH // PSC) + m]
            pltpu.sync_copy(xbuf.at[pl.ds(m * PSC, PSC)], sums_sp.at[idx],
                            add=True)
            pltpu.sync_copy(ones_v, cnt_sp.at[idx], add=True)
        return ()

    lax.fori_loop(0, PRT // PCH, piter, ())
    plsc.subcore_barrier()

    pltpu.sync_copy(sums_sp.at[pl.ds(sid * 33, 33)],
                    sums_o.at[cid, pl.ds(sid * 33, 33)])

    @pl.when(sid == 0)
    def _():
        pltpu.sync_copy(cnt_sp, cnt_o.at[cid])


@functools.cache
def _pool_kernel():
  return pl.kernel(
    _pool_body,
    out_type=[_shape_f((2, GP, H)), _shape_f((2, GP))],
    mesh=plsc.VectorSubcoreMesh(core_axis_name="c", subcore_axis_name="s"),
    compiler_params=pltpu.CompilerParams(use_tc_tiling_on_sc=False),
    scratch_types=[
        pltpu.VMEM((PRT // PSC, PSC), _i32),
        pltpu.VMEM((PCH, H), _f32),
        pltpu.VMEM((PSC,), _f32),
        pltpu.VMEM((33, H), _f32),
        pltpu.VMEM((GP,), _f32),
        pltpu.VMEM_SHARED((GP, H), _f32),
        pltpu.VMEM_SHARED((GP,), _f32),
        pltpu.SemaphoreType.DMA,
    ],
  )


def kernel(x, edge_index, batch, W1, as1, ad1, b1, W2, as2, ad2, b2,
           W3, as3, ad3, b3, Wl, bl):
    f32 = jnp.float32
    xp = jnp.zeros((NP, 24), f32).at[:N, :20].set(x)
    pad = jnp.full((EP - E,), N, jnp.int32)
    src_p = jnp.concatenate([edge_index[0], pad])
    dst_p = jnp.concatenate([edge_index[1], pad])
    dst2 = dst_p.reshape(EP // 128, 128)
    batch2 = jnp.concatenate(
        [batch, jnp.full((NP - N,), G, jnp.int32)]).reshape(NP // PSC, PSC)

    w1p = jnp.zeros((24, H), f32).at[:20].set(W1)

    def flat(a):
        return a.reshape(NP)

    hA, hB, asr, ads, asel = _dense1(xp, w1p, as1.reshape(1, H),
                                     ad1.reshape(1, H))
    accA, accB, den = _edge_kernel()(src_p, dst2, flat(asr), flat(ads),
                                     flat(asel), hA, hB)

    for (Wn, an, dn, bn) in ((W2, as2, ad2, b1), (W3, as3, ad3, b2)):
        hA, hB, asr, ads, asel = _combine_relu(
            accA, accB, den[0].reshape(NP, 1), den[1].reshape(NP, 1),
            hA, hB, bn.reshape(1, H),
            Wn, an.reshape(1, H), dn.reshape(1, H))
        accA, accB, den = _edge_kernel()(src_p, dst2, flat(asr),
                                         flat(ads), flat(asel), hA, hB)

    x3 = _final_combine(accA, accB, den[0].reshape(NP, 1),
                        den[1].reshape(NP, 1), hA, hB, b3.reshape(1, H))
    sums, cnt = _pool_kernel()(x3, batch2)
    return _head(sums, cnt, Wl, bl.reshape(1, 2))


# speedup vs baseline: 1.1170x; 1.1170x over previous
import functools

import jax
import jax.numpy as jnp
from jax import lax
from jax.experimental import pallas as pl
from jax.experimental.pallas import tpu as pltpu
from jax.experimental.pallas import tpu_sc as plsc

N = 50000
NP = 50176
E = 800000
EP = 819200
H = 64
HH = 32
G = 512
GP = 528
RPT = NP // 16
EPT = EP // 16
C = 512
NCH = EPT // C
RB = 1024
NBLK = NP // RB


def _flat_spec():
    return pl.BlockSpec((8, 128), lambda i: (i, 0))


def _attention_rows(h, av, ad):
    asr = jnp.sum(h * av, axis=-1)
    ads = jnp.sum(h * ad, axis=-1)
    t = asr + ads
    asel = jnp.where(t >= 0, t, 0.2 * t)
    return asr, ads, asel


def _dense1_body(x_ref, w_ref, av_ref, ad_ref,
                 hA_ref, hB_ref, as_ref, ad2_ref, asel_ref):
    h = jnp.dot(x_ref[...], w_ref[...], preferred_element_type=jnp.float32)
    hA_ref[...] = h[:, :HH]
    hB_ref[...] = h[:, HH:]
    asr, ads, asel = _attention_rows(h, av_ref[...], ad_ref[...])
    as_ref[...] = asr.reshape(8, 128)
    ad2_ref[...] = ads.reshape(8, 128)
    asel_ref[...] = asel.reshape(8, 128)


def _combine_body(relu, accA_ref, accB_ref, den_ref, hA_ref, hB_ref, b_ref,
                  w_ref, av_ref, ad_ref,
                  hA2_ref, hB2_ref, as_ref, ad2_ref, asel_ref):
    r = 1.0 / (den_ref[...] + 1.0)
    xin = jnp.concatenate(
        [accA_ref[...] + hA_ref[...], accB_ref[...] + hB_ref[...]], axis=1)
    xin = xin * r + b_ref[...]
    if relu:
        xin = jnp.maximum(xin, 0.0)
    h = jnp.dot(xin, w_ref[...], preferred_element_type=jnp.float32)
    hA2_ref[...] = h[:, :HH]
    hB2_ref[...] = h[:, HH:]
    asr, ads, asel = _attention_rows(h, av_ref[...], ad_ref[...])
    as_ref[...] = asr.reshape(8, 128)
    ad2_ref[...] = ads.reshape(8, 128)
    asel_ref[...] = asel.reshape(8, 128)


def _head_body(sums_ref, cnt_ref, b3_ref, wl_ref, bl_ref, out_ref):
    s = sums_ref[0] + sums_ref[1]
    c = cnt_ref[0] + cnt_ref[1]
    pooled = s[:G] / jnp.maximum(c[:G], 1.0)[:, None] + b3_ref[...]
    out_ref[...] = (
        jnp.dot(pooled, wl_ref[...], preferred_element_type=jnp.float32)
        + bl_ref[...])


_half_spec = pl.BlockSpec((RB, HH), lambda i: (i, 0))
_full_row_spec = pl.BlockSpec((RB, H), lambda i: (i, 0))
_vec64_spec = pl.BlockSpec((1, H), lambda i: (0, 0))
_den_spec = pl.BlockSpec((RB, 1), lambda i: (i, 0))

_f32 = jnp.float32


def _shape_f(shape):
    return jax.ShapeDtypeStruct(shape, _f32)


_dense1 = pl.pallas_call(
    _dense1_body,
    grid=(NBLK,),
    in_specs=[
        pl.BlockSpec((RB, 24), lambda i: (i, 0)),
        pl.BlockSpec((24, H), lambda i: (0, 0)),
        _vec64_spec, _vec64_spec,
    ],
    out_specs=[_half_spec, _half_spec, _flat_spec(), _flat_spec(), _flat_spec()],
    out_shape=[_shape_f((NP, HH)), _shape_f((NP, HH)),
               _shape_f((392, 128)), _shape_f((392, 128)), _shape_f((392, 128))],
)


def _make_combine(relu):
    return pl.pallas_call(
        functools.partial(_combine_body, relu),
        grid=(NBLK,),
        in_specs=[
            _half_spec, _half_spec, _den_spec, _half_spec, _half_spec,
            _vec64_spec,
            pl.BlockSpec((H, H), lambda i: (0, 0)),
            _vec64_spec, _vec64_spec,
        ],
        out_specs=[_half_spec, _half_spec, _flat_spec(), _flat_spec(),
                   _flat_spec()],
        out_shape=[_shape_f((NP, HH)), _shape_f((NP, HH)),
                   _shape_f((392, 128)), _shape_f((392, 128)),
                   _shape_f((392, 128))],
    )


_combine_relu = _make_combine(True)

_head = pl.pallas_call(
    _head_body,
    grid=(1,),
    in_specs=[
        pl.BlockSpec((2, GP, H), lambda i: (0, 0, 0)),
        pl.BlockSpec((2, GP), lambda i: (0, 0)),
        _vec64_spec,
        pl.BlockSpec((H, 2), lambda i: (0, 0)),
        pl.BlockSpec((1, 2), lambda i: (0, 0)),
    ],
    out_specs=pl.BlockSpec((G, 2), lambda i: (0, 0)),
    out_shape=_shape_f((G, 2)),
)


_i32 = jnp.int32
_lane16 = jnp.arange(16, dtype=jnp.int32)


def _edge_body(src1, dst2, asrc, adst, aself, hA, hB,
               accA_o, accB_o, den_o,
               src_a, src_b, d2_a, d2_b, av, bv, cv, s1, h_v, zrows, zflat,
               acc_sp, den_sp, sem_ld, sem_g, sem_h, sem_sc):
    cid = lax.axis_index("c")
    wid = lax.axis_index("s")

    @plsc.parallel_loop(0, 64)
    def _(i):
        z = jnp.zeros((16,), _f32)
        zrows[i, pl.ds(0, 16)] = z
        zrows[i, pl.ds(16, 16)] = z

    @plsc.parallel_loop(0, RPT // 16)
    def _(i):
        zflat[pl.ds(i * 16, 16)] = jnp.zeros((16,), _f32)

    row0 = wid * RPT
    for k in range(RPT // 64):
        pltpu.sync_copy(zrows, acc_sp.at[pl.ds(row0 + k * 64, 64)])
    pltpu.sync_copy(zflat, den_sp.at[pl.ds(row0, RPT)])
    plsc.subcore_barrier()

    def jbase(j):
        return pl.multiple_of(wid * EPT + j * C, C)

    def rbase(j):
        return wid * (EPT // 128) + j * (C // 128)

    def drain_scatters(d2_c):
        for jj in range(C // 128):
            pltpu.make_async_copy(h_v.at[pl.ds(jj * 128, 128)],
                                  acc_sp.at[d2_c.at[jj]], sem_sc).wait()

        @pl.when(cid == 0)
        def _():
            for jj in range(C // 128):
                pltpu.make_async_copy(s1.at[pl.ds(jj * 128, 128)],
                                      den_sp.at[d2_c.at[jj]], sem_sc).wait()

    def phase(j, src_c, d2_c, src_n, d2_n, first):
        pltpu.make_async_copy(src1.at[pl.ds(jbase(j), C)], src_c,
                              sem_ld).wait()
        pltpu.make_async_copy(dst2.at[pl.ds(rbase(j), C // 128)], d2_c,
                              sem_ld).wait()
        gw = [pltpu.async_copy(asrc.at[src_c], av, sem_g)]
        for jj in range(C // 128):
            idx = d2_c.at[jj]
            gw.append(pltpu.async_copy(adst.at[idx], bv.at[jj], sem_g))
            gw.append(pltpu.async_copy(aself.at[idx], cv.at[jj], sem_g))

        if first:
            @pl.when(j > 0)
            def _():
                drain_scatters(d2_c)
        else:
            drain_scatters(d2_c)

        @pl.when(cid == 0)
        def _():
            pltpu.async_copy(hA.at[src_c], h_v, sem_h)

        @pl.when(cid == 1)
        def _():
            pltpu.async_copy(hB.at[src_c], h_v, sem_h)

        @pl.when(j + 1 < NCH)
        def _():
            pltpu.async_copy(src1.at[pl.ds(jbase(j + 1), C)], src_n, sem_ld)
            pltpu.async_copy(dst2.at[pl.ds(rbase(j + 1), C // 128)], d2_n,
                             sem_ld)

        for wcp in gw:
            wcp.wait()

        @plsc.parallel_loop(0, C // 16)
        def _(g):
            a16 = av[pl.ds(g * 16, 16)]
            b16 = bv[g // 8, pl.ds((g % 8) * 16, 16)]
            c16 = cv[g // 8, pl.ds((g % 8) * 16, 16)]
            al = a16 + b16
            al = jnp.where(al >= 0, al, 0.2 * al)
            s1[pl.ds(g * 16, 16)] = jnp.exp(jnp.minimum(al - c16, 60.0))

        pltpu.make_async_copy(hA.at[src_c], h_v, sem_h).wait()

        @plsc.parallel_loop(0, C // 16, unroll=2)
        def _(g):
            sval = s1[pl.ds(g * 16, 16)]
            for k in range(16):
                e = g * 16 + k
                sv = sval[k]
                h_v[e, pl.ds(0, 16)] = h_v[e, pl.ds(0, 16)] * sv
                h_v[e, pl.ds(16, 16)] = h_v[e, pl.ds(16, 16)] * sv

        for jj in range(C // 128):
            pltpu.async_copy(h_v.at[pl.ds(jj * 128, 128)],
                             acc_sp.at[d2_c.at[jj]], sem_sc, add=True)

        @pl.when(cid == 0)
        def _():
            for jj in range(C // 128):
                pltpu.async_copy(s1.at[pl.ds(jj * 128, 128)],
                                 den_sp.at[d2_c.at[jj]], sem_sc, add=True)

    pltpu.async_copy(src1.at[pl.ds(jbase(0), C)], src_a, sem_ld)
    pltpu.async_copy(dst2.at[pl.ds(rbase(0), C // 128)], d2_a, sem_ld)

    def pair(t, _):
        phase(2 * t, src_a, d2_a, src_b, d2_b, True)
        phase(2 * t + 1, src_b, d2_b, src_a, d2_a, False)
        return ()

    lax.fori_loop(0, NCH // 2, pair, ())
    drain_scatters(d2_a)
    plsc.subcore_barrier()

    rows = pl.ds(row0, RPT)

    @pl.when(cid == 0)
    def _():
        pltpu.sync_copy(acc_sp.at[rows], accA_o.at[rows])
        pltpu.sync_copy(den_sp.at[rows], den_o.at[rows])

    @pl.when(cid == 1)
    def _():
        pltpu.sync_copy(acc_sp.at[rows], accB_o.at[rows])


@functools.cache
def _edge_kernel():
  return pl.kernel(
    _edge_body,
    out_type=[_shape_f((NP, HH)), _shape_f((NP, HH)), _shape_f((NP,))],
    mesh=plsc.VectorSubcoreMesh(core_axis_name="c", subcore_axis_name="s"),
    compiler_params=pltpu.CompilerParams(use_tc_tiling_on_sc=False),
    scratch_types=[
        pltpu.VMEM((C,), _i32),
        pltpu.VMEM((C,), _i32),
        pltpu.VMEM((C // 128, 128), _i32),
        pltpu.VMEM((C // 128, 128), _i32),
        pltpu.VMEM((C,), _f32),
        pltpu.VMEM((C // 128, 128), _f32),
        pltpu.VMEM((C // 128, 128), _f32),
        pltpu.VMEM((C,), _f32),
        pltpu.VMEM((C, HH), _f32),
        pltpu.VMEM((64, HH), _f32),
        pltpu.VMEM((RPT,), _f32),
        pltpu.VMEM_SHARED((NP, HH), _f32),
        pltpu.VMEM_SHARED((NP,), _f32),
        pltpu.SemaphoreType.DMA,
        pltpu.SemaphoreType.DMA,
        pltpu.SemaphoreType.DMA,
        pltpu.SemaphoreType.DMA,
    ],
  )


PRT = NP // 32
PCH = 224
PSC = 32


def _pool_body(accA, accB, den, hA, hB, b2, sums_o, cnt_o,
               bidx_v, abuf, bbuf, habuf, hbbuf, den_v, xbuf, ones_v,
               zrow, zc, sums_sp, cnt_sp, sem):
    cid = lax.axis_index("c")
    sid = lax.axis_index("s")
    wid = cid * 16 + sid

    @plsc.parallel_loop(0, 2)
    def _(i):
        ones_v[pl.ds(i * 16, 16)] = jnp.full((16,), 1.0, _f32)

    @pl.when(sid == 0)
    def _():
        @plsc.parallel_loop(0, 33 * 4)
        def _(i):
            zrow[i // 4, pl.ds((i % 4) * 16, 16)] = jnp.zeros((16,), _f32)

        @plsc.parallel_loop(0, GP // 16)
        def _(i):
            zc[pl.ds(i * 16, 16)] = jnp.zeros((16,), _f32)

        for k in range(GP // 33):
            pltpu.sync_copy(zrow, sums_sp.at[pl.ds(k * 33, 33)])
        pltpu.sync_copy(zc, cnt_sp)

    plsc.subcore_barrier()

    pltpu.sync_copy(b2.at[pl.ds(wid * (PRT // PSC), PRT // PSC)], bidx_v)

    def piter(k, _):
        base = pl.multiple_of(wid * PRT + k * PCH, PCH)
        rows = pl.ds(base, PCH)
        cps = [pltpu.async_copy(accA.at[rows], abuf, sem),
               pltpu.async_copy(accB.at[rows], bbuf, sem),
               pltpu.async_copy(hA.at[rows], habuf, sem),
               pltpu.async_copy(hB.at[rows], hbbuf, sem),
               pltpu.async_copy(den.at[rows], den_v, sem)]
        for cp in cps:
            cp.wait()

        @plsc.parallel_loop(0, PCH // 16)
        def _(g):
            d16 = den_v[pl.ds(g * 16, 16)]
            rr = 1.0 / (d16 + 1.0)
            for kk in range(16):
                row = g * 16 + kk
                rv = rr[kk]
                xbuf[row, pl.ds(0, 16)] = (
                    abuf[row, pl.ds(0, 16)] + habuf[row, pl.ds(0, 16)]) * rv
                xbuf[row, pl.ds(16, 16)] = (
                    abuf[row, pl.ds(16, 16)] + habuf[row, pl.ds(16, 16)]) * rv
                xbuf[row, pl.ds(32, 16)] = (
                    bbuf[row, pl.ds(0, 16)] + hbbuf[row, pl.ds(0, 16)]) * rv
                xbuf[row, pl.ds(48, 16)] = (
                    bbuf[row, pl.ds(16, 16)] + hbbuf[row, pl.ds(16, 16)]) * rv

        for m in range(PCH // PSC):
            idx = bidx_v.at[k * (PCH // PSC) + m]
            pltpu.sync_copy(xbuf.at[pl.ds(m * PSC, PSC)], sums_sp.at[idx],
                            add=True)
            pltpu.sync_copy(ones_v, cnt_sp.at[idx], add=True)
        return ()

    lax.fori_loop(0, PRT // PCH, piter, ())
    plsc.subcore_barrier()

    pltpu.sync_copy(sums_sp.at[pl.ds(sid * 33, 33)],
                    sums_o.at[cid, pl.ds(sid * 33, 33)])

    @pl.when(sid == 0)
    def _():
        pltpu.sync_copy(cnt_sp, cnt_o.at[cid])


@functools.cache
def _pool_kernel():
  return pl.kernel(
    _pool_body,
    out_type=[_shape_f((2, GP, H)), _shape_f((2, GP))],
    mesh=plsc.VectorSubcoreMesh(core_axis_name="c", subcore_axis_name="s"),
    compiler_params=pltpu.CompilerParams(use_tc_tiling_on_sc=False),
    scratch_types=[
        pltpu.VMEM((PRT // PSC, PSC), _i32),
        pltpu.VMEM((PCH, HH), _f32),
        pltpu.VMEM((PCH, HH), _f32),
        pltpu.VMEM((PCH, HH), _f32),
        pltpu.VMEM((PCH, HH), _f32),
        pltpu.VMEM((PCH,), _f32),
        pltpu.VMEM((PCH, H), _f32),
        pltpu.VMEM((PSC,), _f32),
        pltpu.VMEM((33, H), _f32),
        pltpu.VMEM((GP,), _f32),
        pltpu.VMEM_SHARED((GP, H), _f32),
        pltpu.VMEM_SHARED((GP,), _f32),
        pltpu.SemaphoreType.DMA,
    ],
  )


def kernel(x, edge_index, batch, W1, as1, ad1, b1, W2, as2, ad2, b2,
           W3, as3, ad3, b3, Wl, bl):
    f32 = jnp.float32
    xp = jnp.zeros((NP, 24), f32).at[:N, :20].set(x)
    pad = jnp.full((EP - E,), N, jnp.int32)
    src_p = jnp.concatenate([edge_index[0], pad])
    dst_p = jnp.concatenate([edge_index[1], pad])
    dst2 = dst_p.reshape(EP // 128, 128)
    batch2 = jnp.concatenate(
        [batch, jnp.full((NP - N,), G, jnp.int32)]).reshape(NP // PSC, PSC)

    w1p = jnp.zeros((24, H), f32).at[:20].set(W1)

    def flat(a):
        return a.reshape(NP)

    hA, hB, asr, ads, asel = _dense1(xp, w1p, as1.reshape(1, H),
                                     ad1.reshape(1, H))
    accA, accB, den = _edge_kernel()(src_p, dst2, flat(asr), flat(ads),
                                     flat(asel), hA, hB)

    for (Wn, an, dn, bn) in ((W2, as2, ad2, b1), (W3, as3, ad3, b2)):
        hA, hB, asr, ads, asel = _combine_relu(
            accA, accB, den.reshape(NP, 1), hA, hB, bn.reshape(1, H),
            Wn, an.reshape(1, H), dn.reshape(1, H))
        accA, accB, den = _edge_kernel()(src_p, dst2, flat(asr),
                                         flat(ads), flat(asel), hA, hB)

    sums, cnt = _pool_kernel()(accA, accB, den, hA, hB, batch2)
    return _head(sums, cnt, b3.reshape(1, H), Wl, bl.reshape(1, 2))


# speedup vs baseline: 1.1208x; 1.0034x over previous
import functools

import jax
import jax.numpy as jnp
from jax import lax
from jax.experimental import pallas as pl
from jax.experimental.pallas import tpu as pltpu
from jax.experimental.pallas import tpu_sc as plsc

N = 50000
NP = 50176
E = 800000
EP = 819200
H = 64
HH = 32
G = 512
GP = 528
RPT = NP // 16
EPT = EP // 16
C = 640
NCH = EPT // C
RB = 1024
NBLK = NP // RB


def _flat_spec():
    return pl.BlockSpec((8, 128), lambda i: (i, 0))


def _attention_rows(h, av, ad):
    asr = jnp.sum(h * av, axis=-1)
    ads = jnp.sum(h * ad, axis=-1)
    t = asr + ads
    asel = jnp.where(t >= 0, t, 0.2 * t)
    return asr, ads, asel


def _dense1_body(x_ref, w_ref, av_ref, ad_ref,
                 hA_ref, hB_ref, as_ref, ad2_ref, asel_ref):
    h = jnp.dot(x_ref[...], w_ref[...], preferred_element_type=jnp.float32)
    hA_ref[...] = h[:, :HH]
    hB_ref[...] = h[:, HH:]
    asr, ads, asel = _attention_rows(h, av_ref[...], ad_ref[...])
    as_ref[...] = asr.reshape(8, 128)
    ad2_ref[...] = ads.reshape(8, 128)
    asel_ref[...] = asel.reshape(8, 128)


def _combine_body(relu, accA_ref, accB_ref, den_ref, hA_ref, hB_ref, b_ref,
                  w_ref, av_ref, ad_ref,
                  hA2_ref, hB2_ref, as_ref, ad2_ref, asel_ref):
    r = 1.0 / (den_ref[...] + 1.0)
    xin = jnp.concatenate(
        [accA_ref[...] + hA_ref[...], accB_ref[...] + hB_ref[...]], axis=1)
    xin = xin * r + b_ref[...]
    if relu:
        xin = jnp.maximum(xin, 0.0)
    h = jnp.dot(xin, w_ref[...], preferred_element_type=jnp.float32)
    hA2_ref[...] = h[:, :HH]
    hB2_ref[...] = h[:, HH:]
    asr, ads, asel = _attention_rows(h, av_ref[...], ad_ref[...])
    as_ref[...] = asr.reshape(8, 128)
    ad2_ref[...] = ads.reshape(8, 128)
    asel_ref[...] = asel.reshape(8, 128)


def _head_body(sums_ref, cnt_ref, b3_ref, wl_ref, bl_ref, out_ref):
    s = sums_ref[0] + sums_ref[1]
    c = cnt_ref[0] + cnt_ref[1]
    pooled = s[:G] / jnp.maximum(c[:G], 1.0)[:, None] + b3_ref[...]
    out_ref[...] = (
        jnp.dot(pooled, wl_ref[...], preferred_element_type=jnp.float32)
        + bl_ref[...])


_half_spec = pl.BlockSpec((RB, HH), lambda i: (i, 0))
_full_row_spec = pl.BlockSpec((RB, H), lambda i: (i, 0))
_vec64_spec = pl.BlockSpec((1, H), lambda i: (0, 0))
_den_spec = pl.BlockSpec((RB, 1), lambda i: (i, 0))

_f32 = jnp.float32


def _shape_f(shape):
    return jax.ShapeDtypeStruct(shape, _f32)


_dense1 = pl.pallas_call(
    _dense1_body,
    grid=(NBLK,),
    in_specs=[
        pl.BlockSpec((RB, 24), lambda i: (i, 0)),
        pl.BlockSpec((24, H), lambda i: (0, 0)),
        _vec64_spec, _vec64_spec,
    ],
    out_specs=[_half_spec, _half_spec, _flat_spec(), _flat_spec(), _flat_spec()],
    out_shape=[_shape_f((NP, HH)), _shape_f((NP, HH)),
               _shape_f((392, 128)), _shape_f((392, 128)), _shape_f((392, 128))],
)


def _make_combine(relu):
    return pl.pallas_call(
        functools.partial(_combine_body, relu),
        grid=(NBLK,),
        in_specs=[
            _half_spec, _half_spec, _den_spec, _half_spec, _half_spec,
            _vec64_spec,
            pl.BlockSpec((H, H), lambda i: (0, 0)),
            _vec64_spec, _vec64_spec,
        ],
        out_specs=[_half_spec, _half_spec, _flat_spec(), _flat_spec(),
                   _flat_spec()],
        out_shape=[_shape_f((NP, HH)), _shape_f((NP, HH)),
                   _shape_f((392, 128)), _shape_f((392, 128)),
                   _shape_f((392, 128))],
    )


_combine_relu = _make_combine(True)

_head = pl.pallas_call(
    _head_body,
    grid=(1,),
    in_specs=[
        pl.BlockSpec((2, GP, H), lambda i: (0, 0, 0)),
        pl.BlockSpec((2, GP), lambda i: (0, 0)),
        _vec64_spec,
        pl.BlockSpec((H, 2), lambda i: (0, 0)),
        pl.BlockSpec((1, 2), lambda i: (0, 0)),
    ],
    out_specs=pl.BlockSpec((G, 2), lambda i: (0, 0)),
    out_shape=_shape_f((G, 2)),
)


_i32 = jnp.int32
_lane16 = jnp.arange(16, dtype=jnp.int32)


def _edge_body(src1, dst2, asrc, adst, aself, hA, hB,
               accA_o, accB_o, den_o,
               src_a, src_b, d2_a, d2_b, av, bv, cv, s1, h_v, zrows, zflat,
               acc_sp, den_sp, sem_ld, sem_g, sem_h, sem_sc):
    cid = lax.axis_index("c")
    wid = lax.axis_index("s")

    @plsc.parallel_loop(0, 32)
    def _(i):
        z = jnp.zeros((16,), _f32)
        zrows[i, pl.ds(0, 16)] = z
        zrows[i, pl.ds(16, 16)] = z

    @plsc.parallel_loop(0, 448 // 16)
    def _(i):
        zflat[pl.ds(i * 16, 16)] = jnp.zeros((16,), _f32)

    row0 = wid * RPT
    for k in range(RPT // 32):
        pltpu.sync_copy(zrows, acc_sp.at[pl.ds(row0 + k * 32, 32)])
    for k in range(RPT // 448):
        pltpu.sync_copy(zflat, den_sp.at[pl.ds(row0 + k * 448, 448)])
    plsc.subcore_barrier()

    def jbase(j):
        return pl.multiple_of(wid * EPT + j * C, C)

    def rbase(j):
        return wid * (EPT // 128) + j * (C // 128)

    def drain_scatters(d2_c):
        for jj in range(C // 128):
            pltpu.make_async_copy(h_v.at[pl.ds(jj * 128, 128)],
                                  acc_sp.at[d2_c.at[jj]], sem_sc).wait()

        @pl.when(cid == 0)
        def _():
            for jj in range(C // 128):
                pltpu.make_async_copy(s1.at[pl.ds(jj * 128, 128)],
                                      den_sp.at[d2_c.at[jj]], sem_sc).wait()

    def phase(j, src_c, d2_c, src_n, d2_n, first):
        pltpu.make_async_copy(src1.at[pl.ds(jbase(j), C)], src_c,
                              sem_ld).wait()
        pltpu.make_async_copy(dst2.at[pl.ds(rbase(j), C // 128)], d2_c,
                              sem_ld).wait()
        gw = [pltpu.async_copy(asrc.at[src_c], av, sem_g)]
        for jj in range(C // 128):
            idx = d2_c.at[jj]
            gw.append(pltpu.async_copy(adst.at[idx], bv.at[jj], sem_g))
            gw.append(pltpu.async_copy(aself.at[idx], cv.at[jj], sem_g))

        if first:
            @pl.when(j > 0)
            def _():
                drain_scatters(d2_c)
        else:
            drain_scatters(d2_c)

        @pl.when(cid == 0)
        def _():
            pltpu.async_copy(hA.at[src_c], h_v, sem_h)

        @pl.when(cid == 1)
        def _():
            pltpu.async_copy(hB.at[src_c], h_v, sem_h)

        @pl.when(j + 1 < NCH)
        def _():
            pltpu.async_copy(src1.at[pl.ds(jbase(j + 1), C)], src_n, sem_ld)
            pltpu.async_copy(dst2.at[pl.ds(rbase(j + 1), C // 128)], d2_n,
                             sem_ld)

        for wcp in gw:
            wcp.wait()

        @plsc.parallel_loop(0, C // 16)
        def _(g):
            a16 = av[pl.ds(g * 16, 16)]
            b16 = bv[g // 8, pl.ds((g % 8) * 16, 16)]
            c16 = cv[g // 8, pl.ds((g % 8) * 16, 16)]
            al = a16 + b16
            al = jnp.where(al >= 0, al, 0.2 * al)
            s1[pl.ds(g * 16, 16)] = jnp.exp(jnp.minimum(al - c16, 60.0))

        pltpu.make_async_copy(hA.at[src_c], h_v, sem_h).wait()

        @plsc.parallel_loop(0, C // 16, unroll=2)
        def _(g):
            sval = s1[pl.ds(g * 16, 16)]
            for k in range(16):
                e = g * 16 + k
                sv = sval[k]
                h_v[e, pl.ds(0, 16)] = h_v[e, pl.ds(0, 16)] * sv
                h_v[e, pl.ds(16, 16)] = h_v[e, pl.ds(16, 16)] * sv

        for jj in range(C // 128):
            pltpu.async_copy(h_v.at[pl.ds(jj * 128, 128)],
                             acc_sp.at[d2_c.at[jj]], sem_sc, add=True)

        @pl.when(cid == 0)
        def _():
            for jj in range(C // 128):
                pltpu.async_copy(s1.at[pl.ds(jj * 128, 128)],
                                 den_sp.at[d2_c.at[jj]], sem_sc, add=True)

    pltpu.async_copy(src1.at[pl.ds(jbase(0), C)], src_a, sem_ld)
    pltpu.async_copy(dst2.at[pl.ds(rbase(0), C // 128)], d2_a, sem_ld)

    def pair(t, _):
        phase(2 * t, src_a, d2_a, src_b, d2_b, True)
        phase(2 * t + 1, src_b, d2_b, src_a, d2_a, False)
        return ()

    lax.fori_loop(0, NCH // 2, pair, ())
    drain_scatters(d2_a)
    plsc.subcore_barrier()

    rows = pl.ds(row0, RPT)

    @pl.when(cid == 0)
    def _():
        pltpu.sync_copy(acc_sp.at[rows], accA_o.at[rows])
        pltpu.sync_copy(den_sp.at[rows], den_o.at[rows])

    @pl.when(cid == 1)
    def _():
        pltpu.sync_copy(acc_sp.at[rows], accB_o.at[rows])


@functools.cache
def _edge_kernel():
  return pl.kernel(
    _edge_body,
    out_type=[_shape_f((NP, HH)), _shape_f((NP, HH)), _shape_f((NP,))],
    mesh=plsc.VectorSubcoreMesh(core_axis_name="c", subcore_axis_name="s"),
    compiler_params=pltpu.CompilerParams(use_tc_tiling_on_sc=False),
    scratch_types=[
        pltpu.VMEM((C,), _i32),
        pltpu.VMEM((C,), _i32),
        pltpu.VMEM((C // 128, 128), _i32),
        pltpu.VMEM((C // 128, 128), _i32),
        pltpu.VMEM((C,), _f32),
        pltpu.VMEM((C // 128, 128), _f32),
        pltpu.VMEM((C // 128, 128), _f32),
        pltpu.VMEM((C,), _f32),
        pltpu.VMEM((C, HH), _f32),
        pltpu.VMEM((32, HH), _f32),
        pltpu.VMEM((448,), _f32),
        pltpu.VMEM_SHARED((NP, HH), _f32),
        pltpu.VMEM_SHARED((NP,), _f32),
        pltpu.SemaphoreType.DMA,
        pltpu.SemaphoreType.DMA,
        pltpu.SemaphoreType.DMA,
        pltpu.SemaphoreType.DMA,
    ],
  )


PRT = NP // 32
PCH = 224
PSC = 32


def _pool_body(accA, accB, den, hA, hB, b2, sums_o, cnt_o,
               bidx_v, abuf, bbuf, habuf, hbbuf, den_v, xbuf, ones_v,
               zrow, zc, sums_sp, cnt_sp, sem):
    cid = lax.axis_index("c")
    sid = lax.axis_index("s")
    wid = cid * 16 + sid

    @plsc.parallel_loop(0, 2)
    def _(i):
        ones_v[pl.ds(i * 16, 16)] = jnp.full((16,), 1.0, _f32)

    @pl.when(sid == 0)
    def _():
        @plsc.parallel_loop(0, 33 * 4)
        def _(i):
            zrow[i // 4, pl.ds((i % 4) * 16, 16)] = jnp.zeros((16,), _f32)

        @plsc.parallel_loop(0, GP // 16)
        def _(i):
            zc[pl.ds(i * 16, 16)] = jnp.zeros((16,), _f32)

        for k in range(GP // 33):
            pltpu.sync_copy(zrow, sums_sp.at[pl.ds(k * 33, 33)])
        pltpu.sync_copy(zc, cnt_sp)

    plsc.subcore_barrier()

    pltpu.sync_copy(b2.at[pl.ds(wid * (PRT // PSC), PRT // PSC)], bidx_v)

    def piter(k, _):
        base = pl.multiple_of(wid * PRT + k * PCH, PCH)
        rows = pl.ds(base, PCH)
        cps = [pltpu.async_copy(accA.at[rows], abuf, sem),
               pltpu.async_copy(accB.at[rows], bbuf, sem),
               pltpu.async_copy(hA.at[rows], habuf, sem),
               pltpu.async_copy(hB.at[rows], hbbuf, sem),
               pltpu.async_copy(den.at[rows], den_v, sem)]
        for cp in cps:
            cp.wait()

        @plsc.parallel_loop(0, PCH // 16)
        def _(g):
            d16 = den_v[pl.ds(g * 16, 16)]
            rr = 1.0 / (d16 + 1.0)
            for kk in range(16):
                row = g * 16 + kk
                rv = rr[kk]
                xbuf[row, pl.ds(0, 16)] = (
                    abuf[row, pl.ds(0, 16)] + habuf[row, pl.ds(0, 16)]) * rv
                xbuf[row, pl.ds(16, 16)] = (
                    abuf[row, pl.ds(16, 16)] + habuf[row, pl.ds(16, 16)]) * rv
                xbuf[row, pl.ds(32, 16)] = (
                    bbuf[row, pl.ds(0, 16)] + hbbuf[row, pl.ds(0, 16)]) * rv
                xbuf[row, pl.ds(48, 16)] = (
                    bbuf[row, pl.ds(16, 16)] + hbbuf[row, pl.ds(16, 16)]) * rv

        for m in range(PCH // PSC):
            idx = bidx_v.at[k * (PCH // PSC) + m]
            pltpu.sync_copy(xbuf.at[pl.ds(m * PSC, PSC)], sums_sp.at[idx],
                            add=True)
            pltpu.sync_copy(ones_v, cnt_sp.at[idx], add=True)
        return ()

    lax.fori_loop(0, PRT // PCH, piter, ())
    plsc.subcore_barrier()

    pltpu.sync_copy(sums_sp.at[pl.ds(sid * 33, 33)],
                    sums_o.at[cid, pl.ds(sid * 33, 33)])

    @pl.when(sid == 0)
    def _():
        pltpu.sync_copy(cnt_sp, cnt_o.at[cid])


@functools.cache
def _pool_kernel():
  return pl.kernel(
    _pool_body,
    out_type=[_shape_f((2, GP, H)), _shape_f((2, GP))],
    mesh=plsc.VectorSubcoreMesh(core_axis_name="c", subcore_axis_name="s"),
    compiler_params=pltpu.CompilerParams(use_tc_tiling_on_sc=False),
    scratch_types=[
        pltpu.VMEM((PRT // PSC, PSC), _i32),
        pltpu.VMEM((PCH, HH), _f32),
        pltpu.VMEM((PCH, HH), _f32),
        pltpu.VMEM((PCH, HH), _f32),
        pltpu.VMEM((PCH, HH), _f32),
        pltpu.VMEM((PCH,), _f32),
        pltpu.VMEM((PCH, H), _f32),
        pltpu.VMEM((PSC,), _f32),
        pltpu.VMEM((33, H), _f32),
        pltpu.VMEM((GP,), _f32),
        pltpu.VMEM_SHARED((GP, H), _f32),
        pltpu.VMEM_SHARED((GP,), _f32),
        pltpu.SemaphoreType.DMA,
    ],
  )


def kernel(x, edge_index, batch, W1, as1, ad1, b1, W2, as2, ad2, b2,
           W3, as3, ad3, b3, Wl, bl):
    f32 = jnp.float32
    xp = jnp.zeros((NP, 24), f32).at[:N, :20].set(x)
    pad = jnp.full((EP - E,), N, jnp.int32)
    src_p = jnp.concatenate([edge_index[0], pad])
    dst_p = jnp.concatenate([edge_index[1], pad])
    dst2 = dst_p.reshape(EP // 128, 128)
    batch2 = jnp.concatenate(
        [batch, jnp.full((NP - N,), G, jnp.int32)]).reshape(NP // PSC, PSC)

    w1p = jnp.zeros((24, H), f32).at[:20].set(W1)

    def flat(a):
        return a.reshape(NP)

    hA, hB, asr, ads, asel = _dense1(xp, w1p, as1.reshape(1, H),
                                     ad1.reshape(1, H))
    accA, accB, den = _edge_kernel()(src_p, dst2, flat(asr), flat(ads),
                                     flat(asel), hA, hB)

    for (Wn, an, dn, bn) in ((W2, as2, ad2, b1), (W3, as3, ad3, b2)):
        hA, hB, asr, ads, asel = _combine_relu(
            accA, accB, den.reshape(NP, 1), hA, hB, bn.reshape(1, H),
            Wn, an.reshape(1, H), dn.reshape(1, H))
        accA, accB, den = _edge_kernel()(src_p, dst2, flat(asr),
                                         flat(ads), flat(asel), hA, hB)

    sums, cnt = _pool_kernel()(accA, accB, den, hA, hB, batch2)
    return _head(sums, cnt, b3.reshape(1, H), Wl, bl.reshape(1, 2))
